# Initial kernel scaffold; baseline (speedup 1.0000x reference)
import jax
import jax.numpy as jnp
from jax.experimental import pallas as pl


def kernel(adj_list, x, src_mask, W1, b1, W2, b2, Wt, bt):
    raise NotImplementedError("write your pallas kernel here")


# speedup vs baseline: 1.6284x; 1.6284x over previous
import functools

import jax
import jax.numpy as jnp
from jax.experimental import pallas as pl


def _score_block(adj_ref, x_ref, w1_ref, b1_ref, w2t_ref, b2_ref, out_ref,
                 *, bm):
    i = pl.program_id(2)
    a = adj_ref[0, 0]
    xf = x_ref[0]
    xb = x_ref[0, pl.ds(i * bm, bm), :]
    ax = jnp.dot(a.astype(jnp.bfloat16), xf.astype(jnp.bfloat16),
                 preferred_element_type=jnp.float32)
    denom = jnp.sum(a, axis=1, keepdims=True) + 1.0
    w1 = w1_ref[...].astype(jnp.bfloat16)
    b1 = b1_ref[...]
    axw = (jnp.dot(ax.astype(jnp.bfloat16), w1,
                   preferred_element_type=jnp.float32) + b1) + (
        jnp.dot(xb.astype(jnp.bfloat16), w1,
                preferred_element_type=jnp.float32) + b1)
    axw = axw / denom
    g = jnp.maximum(axw, 0.0)
    gb = g.astype(jnp.bfloat16).astype(jnp.float32)
    w2 = w2t_ref[...].astype(jnp.bfloat16).astype(jnp.float32)
    score = jnp.sum(gb * w2, axis=1, keepdims=True) + b2_ref[0, 0]
    out_ref[0, 0] = score


def _select_block(scores_ref, srcmask_ref, out_ref, *, k, heads, b, n):
    s = scores_ref[...]
    key = jax.lax.bitcast_convert_type(s, jnp.int32)
    key = key ^ jax.lax.shift_right_arithmetic(key, 31) & jnp.int32(0x7FFFFFFF)
    rows = heads * b
    int_min = jnp.int32(-2147483648)
    cnt = jnp.sum((key < 0).astype(jnp.int32), axis=1, keepdims=True)
    t = jnp.where(cnt < k, jnp.zeros((rows, 1), jnp.int32),
                  jnp.full((rows, 1), int_min))
    for bit in range(30, -1, -1):
        cand = t + jnp.int32(1 << bit)
        cnt = jnp.sum((key < cand).astype(jnp.int32), axis=1, keepdims=True)
        t = jnp.where(cnt < k, cand, t)
    lt = key < t
    cnt_lt = jnp.sum(lt.astype(jnp.int32), axis=1, keepdims=True)
    need = k - cnt_lt
    eq = (key == t).astype(jnp.int32)
    col = jax.lax.broadcasted_iota(jnp.int32, (rows, n), 1)
    incl = eq
    sh = 1
    while sh < n:
        incl = incl + jnp.where(col >= sh, jnp.roll(incl, sh, axis=1), 0)
        sh *= 2
    prefix_excl = incl - eq
    sel = lt | ((eq > 0) & (prefix_excl < need))
    sel = sel.astype(jnp.int32)
    union = sel[0:b, :]
    for h in range(1, heads):
        union = union | sel[h * b:(h + 1) * b, :]
    out_ref[...] = jnp.where(union > 0, 0, srcmask_ref[...])


def _xcat_block(x_ref, wt_ref, bt_ref, out_ref, *, heads):
    y = jnp.dot(x_ref[0], wt_ref[...],
                preferred_element_type=jnp.float32) + bt_ref[...]
    out_ref[0] = jnp.concatenate([y] * heads, axis=1)


def kernel(adj_list, x, src_mask, W1, b1, W2, b2, Wt, bt):
    heads, b, n, _ = adj_list.shape
    d = x.shape[-1]
    k = int(0.5 * n) + 1
    bm = min(512, n)
    nb = n // bm

    scores = pl.pallas_call(
        functools.partial(_score_block, bm=bm),
        grid=(heads, b, nb),
        in_specs=[
            pl.BlockSpec((1, 1, bm, n), lambda h, bb, i: (h, bb, i, 0)),
            pl.BlockSpec((1, n, d), lambda h, bb, i: (bb, 0, 0)),
            pl.BlockSpec((d, d), lambda h, bb, i: (0, 0)),
            pl.BlockSpec((1, d), lambda h, bb, i: (0, 0)),
            pl.BlockSpec((1, d), lambda h, bb, i: (0, 0)),
            pl.BlockSpec((1, 1), lambda h, bb, i: (0, 0)),
        ],
        out_specs=pl.BlockSpec((1, 1, bm, 1), lambda h, bb, i: (h, bb, i, 0)),
        out_shape=jax.ShapeDtypeStruct((heads, b, nb * bm, 1), jnp.float32),
    )(adj_list, x, W1, b1.reshape(1, d), W2.reshape(1, d)[:, :],
      b2.reshape(1, 1))

    scores2d = scores.reshape(heads * b, n)
    src2d = src_mask.reshape(b, n).astype(jnp.int32)

    mask2d = pl.pallas_call(
        functools.partial(_select_block, k=k, heads=heads, b=b, n=n),
        out_shape=jax.ShapeDtypeStruct((b, n), jnp.int32),
    )(scores2d, src2d)
    mask_out = mask2d.astype(jnp.bool_).reshape(b, 1, n)

    x_cat = pl.pallas_call(
        functools.partial(_xcat_block, heads=heads),
        grid=(b,),
        in_specs=[
            pl.BlockSpec((1, n, d), lambda bb: (bb, 0, 0)),
            pl.BlockSpec((d, d // heads), lambda bb: (0, 0)),
            pl.BlockSpec((1, d // heads), lambda bb: (0, 0)),
        ],
        out_specs=pl.BlockSpec((1, n, d), lambda bb: (bb, 0, 0)),
        out_shape=jax.ShapeDtypeStruct((b, n, d), jnp.float32),
    )(x, Wt, bt.reshape(1, d // heads))

    return (adj_list, x_cat, mask_out)


# speedup vs baseline: 2.4077x; 1.4786x over previous
import functools

import jax
import jax.numpy as jnp
from jax.experimental import pallas as pl


def _score_block(adj_ref, x_ref, w1_ref, b1_ref, w2t_ref, b2_ref, out_ref,
                 adj_out_ref, *, bm):
    i = pl.program_id(2)
    a = adj_ref[0, 0]
    adj_out_ref[0, 0] = a
    xf = x_ref[0]
    xb = x_ref[0, pl.ds(i * bm, bm), :]
    ax = jnp.dot(a.astype(jnp.bfloat16), xf.astype(jnp.bfloat16),
                 preferred_element_type=jnp.float32)
    denom = jnp.sum(a, axis=1, keepdims=True) + 1.0
    w1 = w1_ref[...].astype(jnp.bfloat16)
    b1 = b1_ref[...]
    axw = (jnp.dot(ax.astype(jnp.bfloat16), w1,
                   preferred_element_type=jnp.float32) + b1) + (
        jnp.dot(xb.astype(jnp.bfloat16), w1,
                preferred_element_type=jnp.float32) + b1)
    axw = axw / denom
    g = jnp.maximum(axw, 0.0)
    gb = g.astype(jnp.bfloat16).astype(jnp.float32)
    w2 = w2t_ref[...].astype(jnp.bfloat16).astype(jnp.float32)
    score = jnp.sum(gb * w2, axis=1, keepdims=True) + b2_ref[0, 0]
    out_ref[0, 0] = score


def _select_block(scores_ref, srcmask_ref, out_ref, *, k, heads, b, n):
    s = scores_ref[...]
    key = jax.lax.bitcast_convert_type(s, jnp.int32)
    key = key ^ jax.lax.shift_right_arithmetic(key, 31) & jnp.int32(0x7FFFFFFF)
    rows = heads * b
    int_min = jnp.int32(-2147483648)
    cnt = jnp.sum((key < 0).astype(jnp.int32), axis=1, keepdims=True)
    t = jnp.where(cnt < k, jnp.zeros((rows, 1), jnp.int32),
                  jnp.full((rows, 1), int_min))
    for bit in range(30, -1, -1):
        cand = t + jnp.int32(1 << bit)
        cnt = jnp.sum((key < cand).astype(jnp.int32), axis=1, keepdims=True)
        t = jnp.where(cnt < k, cand, t)
    lt = key < t
    cnt_lt = jnp.sum(lt.astype(jnp.int32), axis=1, keepdims=True)
    need = k - cnt_lt
    eq = (key == t).astype(jnp.int32)
    col = jax.lax.broadcasted_iota(jnp.int32, (rows, n), 1)
    incl = eq
    sh = 1
    while sh < n:
        incl = incl + jnp.where(col >= sh, jnp.roll(incl, sh, axis=1), 0)
        sh *= 2
    prefix_excl = incl - eq
    sel = lt | ((eq > 0) & (prefix_excl < need))
    sel = sel.astype(jnp.int32)
    union = sel[0:b, :]
    for h in range(1, heads):
        union = union | sel[h * b:(h + 1) * b, :]
    out_ref[...] = jnp.where(union > 0, 0, srcmask_ref[...])


def _xcat_block(x_ref, wt_ref, bt_ref, out_ref, *, heads):
    y = jnp.dot(x_ref[0], wt_ref[...],
                preferred_element_type=jnp.float32) + bt_ref[...]
    out_ref[0] = jnp.concatenate([y] * heads, axis=1)


def kernel(adj_list, x, src_mask, W1, b1, W2, b2, Wt, bt):
    heads, b, n, _ = adj_list.shape
    d = x.shape[-1]
    k = int(0.5 * n) + 1
    bm = min(512, n)
    nb = n // bm

    scores, adj_out = pl.pallas_call(
        functools.partial(_score_block, bm=bm),
        grid=(heads, b, nb),
        in_specs=[
            pl.BlockSpec((1, 1, bm, n), lambda h, bb, i: (h, bb, i, 0)),
            pl.BlockSpec((1, n, d), lambda h, bb, i: (bb, 0, 0)),
            pl.BlockSpec((d, d), lambda h, bb, i: (0, 0)),
            pl.BlockSpec((1, d), lambda h, bb, i: (0, 0)),
            pl.BlockSpec((1, d), lambda h, bb, i: (0, 0)),
            pl.BlockSpec((1, 1), lambda h, bb, i: (0, 0)),
        ],
        out_specs=[
            pl.BlockSpec((1, 1, bm, 1), lambda h, bb, i: (h, bb, i, 0)),
            pl.BlockSpec((1, 1, bm, n), lambda h, bb, i: (h, bb, i, 0)),
        ],
        out_shape=[
            jax.ShapeDtypeStruct((heads, b, nb * bm, 1), jnp.float32),
            jax.ShapeDtypeStruct((heads, b, n, n), jnp.float32),
        ],
    )(adj_list, x, W1, b1.reshape(1, d), W2.reshape(1, d)[:, :],
      b2.reshape(1, 1))

    scores2d = scores.reshape(heads * b, n)
    src2d = src_mask.reshape(b, n).astype(jnp.int32)

    mask2d = pl.pallas_call(
        functools.partial(_select_block, k=k, heads=heads, b=b, n=n),
        out_shape=jax.ShapeDtypeStruct((b, n), jnp.int32),
    )(scores2d, src2d)
    mask_out = mask2d.astype(jnp.bool_).reshape(b, 1, n)

    x_cat = pl.pallas_call(
        functools.partial(_xcat_block, heads=heads),
        grid=(b,),
        in_specs=[
            pl.BlockSpec((1, n, d), lambda bb: (bb, 0, 0)),
            pl.BlockSpec((d, d // heads), lambda bb: (0, 0)),
            pl.BlockSpec((1, d // heads), lambda bb: (0, 0)),
        ],
        out_specs=pl.BlockSpec((1, n, d), lambda bb: (bb, 0, 0)),
        out_shape=jax.ShapeDtypeStruct((b, n, d), jnp.float32),
    )(x, Wt, bt.reshape(1, d // heads))

    return (adj_out, x_cat, mask_out)


# speedup vs baseline: 2.4122x; 1.0019x over previous
import functools

import jax
import jax.numpy as jnp
from jax.experimental import pallas as pl
from jax.experimental.pallas import tpu as pltpu


def _score_block(adj_ref, x_ref, w1_ref, b1_ref, w2t_ref, b2_ref, out_ref,
                 adj_out_ref, *, bm):
    i = pl.program_id(2)
    a = adj_ref[0, 0]
    adj_out_ref[0, 0] = a
    xf = x_ref[0]
    xb = x_ref[0, pl.ds(i * bm, bm), :]
    ax = jnp.dot(a.astype(jnp.bfloat16), xf.astype(jnp.bfloat16),
                 preferred_element_type=jnp.float32)
    denom = jnp.sum(a, axis=1, keepdims=True) + 1.0
    w1 = w1_ref[...].astype(jnp.bfloat16)
    b1 = b1_ref[...]
    axw = (jnp.dot(ax.astype(jnp.bfloat16), w1,
                   preferred_element_type=jnp.float32) + b1) + (
        jnp.dot(xb.astype(jnp.bfloat16), w1,
                preferred_element_type=jnp.float32) + b1)
    axw = axw / denom
    g = jnp.maximum(axw, 0.0)
    gb = g.astype(jnp.bfloat16).astype(jnp.float32)
    w2 = w2t_ref[...].astype(jnp.bfloat16).astype(jnp.float32)
    score = jnp.sum(gb * w2, axis=1, keepdims=True) + b2_ref[0, 0]
    out_ref[0, 0] = score


def _select_block(scores_ref, srcmask_ref, out_ref, *, k, heads, b, n):
    s = scores_ref[...]
    key = jax.lax.bitcast_convert_type(s, jnp.int32)
    key = key ^ jax.lax.shift_right_arithmetic(key, 31) & jnp.int32(0x7FFFFFFF)
    rows = heads * b
    int_min = jnp.int32(-2147483648)
    cnt = jnp.sum((key < 0).astype(jnp.int32), axis=1, keepdims=True)
    t = jnp.where(cnt < k, jnp.zeros((rows, 1), jnp.int32),
                  jnp.full((rows, 1), int_min))
    for bit in range(30, -1, -1):
        cand = t + jnp.int32(1 << bit)
        cnt = jnp.sum((key < cand).astype(jnp.int32), axis=1, keepdims=True)
        t = jnp.where(cnt < k, cand, t)
    lt = key < t
    cnt_lt = jnp.sum(lt.astype(jnp.int32), axis=1, keepdims=True)
    need = k - cnt_lt
    eq = (key == t).astype(jnp.int32)
    col = jax.lax.broadcasted_iota(jnp.int32, (rows, n), 1)
    incl = eq
    sh = 1
    while sh < n:
        incl = incl + jnp.where(col >= sh, jnp.roll(incl, sh, axis=1), 0)
        sh *= 2
    prefix_excl = incl - eq
    sel = lt | ((eq > 0) & (prefix_excl < need))
    sel = sel.astype(jnp.int32)
    union = sel[0:b, :]
    for h in range(1, heads):
        union = union | sel[h * b:(h + 1) * b, :]
    out_ref[...] = jnp.where(union > 0, 0, srcmask_ref[...])


def _xcat_block(x_ref, wt_ref, bt_ref, out_ref, *, heads):
    y = jnp.dot(x_ref[0], wt_ref[...],
                preferred_element_type=jnp.float32) + bt_ref[...]
    out_ref[0] = jnp.concatenate([y] * heads, axis=1)


def kernel(adj_list, x, src_mask, W1, b1, W2, b2, Wt, bt):
    heads, b, n, _ = adj_list.shape
    d = x.shape[-1]
    k = int(0.5 * n) + 1
    bm = min(512, n)
    nb = n // bm

    scores, adj_out = pl.pallas_call(
        functools.partial(_score_block, bm=bm),
        grid=(heads, b, nb),
        in_specs=[
            pl.BlockSpec((1, 1, bm, n), lambda h, bb, i: (h, bb, i, 0)),
            pl.BlockSpec((1, n, d), lambda h, bb, i: (bb, 0, 0)),
            pl.BlockSpec((d, d), lambda h, bb, i: (0, 0)),
            pl.BlockSpec((1, d), lambda h, bb, i: (0, 0)),
            pl.BlockSpec((1, d), lambda h, bb, i: (0, 0)),
            pl.BlockSpec((1, 1), lambda h, bb, i: (0, 0)),
        ],
        out_specs=[
            pl.BlockSpec((1, 1, bm, 1), lambda h, bb, i: (h, bb, i, 0)),
            pl.BlockSpec((1, 1, bm, n), lambda h, bb, i: (h, bb, i, 0)),
        ],
        out_shape=[
            jax.ShapeDtypeStruct((heads, b, nb * bm, 1), jnp.float32),
            jax.ShapeDtypeStruct((heads, b, n, n), jnp.float32),
        ],
        compiler_params=pltpu.CompilerParams(
            dimension_semantics=("parallel", "parallel", "arbitrary")),
    )(adj_list, x, W1, b1.reshape(1, d), W2.reshape(1, d)[:, :],
      b2.reshape(1, 1))

    scores2d = scores.reshape(heads * b, n)
    src2d = src_mask.reshape(b, n).astype(jnp.int32)

    mask2d = pl.pallas_call(
        functools.partial(_select_block, k=k, heads=heads, b=b, n=n),
        out_shape=jax.ShapeDtypeStruct((b, n), jnp.int32),
    )(scores2d, src2d)
    mask_out = mask2d.astype(jnp.bool_).reshape(b, 1, n)

    x_cat = pl.pallas_call(
        functools.partial(_xcat_block, heads=heads),
        grid=(b,),
        in_specs=[
            pl.BlockSpec((1, n, d), lambda bb: (bb, 0, 0)),
            pl.BlockSpec((d, d // heads), lambda bb: (0, 0)),
            pl.BlockSpec((1, d // heads), lambda bb: (0, 0)),
        ],
        out_specs=pl.BlockSpec((1, n, d), lambda bb: (bb, 0, 0)),
        out_shape=jax.ShapeDtypeStruct((b, n, d), jnp.float32),
    )(x, Wt, bt.reshape(1, d // heads))

    return (adj_out, x_cat, mask_out)


# speedup vs baseline: 2.4674x; 1.0229x over previous
import functools

import jax
import jax.numpy as jnp
from jax.experimental import pallas as pl
from jax.experimental.pallas import tpu as pltpu


def _score_block(adj_ref, x_ref, w1_ref, b1_ref, w2t_ref, b2_ref, out_ref,
                 adj_out_ref, *, bm):
    i = pl.program_id(2)
    a = adj_ref[0, 0]
    adj_out_ref[0, 0] = a
    xf = x_ref[0]
    xb = x_ref[0, pl.ds(i * bm, bm), :]
    ax = jnp.dot(a.astype(jnp.bfloat16), xf.astype(jnp.bfloat16),
                 preferred_element_type=jnp.float32)
    denom = jnp.sum(a, axis=1, keepdims=True) + 1.0
    w1 = w1_ref[...].astype(jnp.bfloat16)
    b1 = b1_ref[...]
    axw = (jnp.dot(ax.astype(jnp.bfloat16), w1,
                   preferred_element_type=jnp.float32) + b1) + (
        jnp.dot(xb.astype(jnp.bfloat16), w1,
                preferred_element_type=jnp.float32) + b1)
    axw = axw / denom
    g = jnp.maximum(axw, 0.0)
    gb = g.astype(jnp.bfloat16).astype(jnp.float32)
    w2 = w2t_ref[...].astype(jnp.bfloat16).astype(jnp.float32)
    score = jnp.sum(gb * w2, axis=1, keepdims=True) + b2_ref[0, 0]
    out_ref[0, 0] = score


def _select_block(scores_ref, srcmask_ref, out_ref, *, k, heads, b, n):
    s = scores_ref[...]
    key = jax.lax.bitcast_convert_type(s, jnp.int32)
    key = key ^ jax.lax.shift_right_arithmetic(key, 31) & jnp.int32(0x7FFFFFFF)
    rows = heads * b
    int_min = jnp.int32(-2147483648)
    cnt = jnp.sum((key < 0).astype(jnp.int32), axis=1, keepdims=True)
    t = jnp.where(cnt < k, jnp.zeros((rows, 1), jnp.int32),
                  jnp.full((rows, 1), int_min))
    for bit in range(30, -1, -1):
        cand = t + jnp.int32(1 << bit)
        cnt = jnp.sum((key < cand).astype(jnp.int32), axis=1, keepdims=True)
        t = jnp.where(cnt < k, cand, t)
    lt = key < t
    cnt_lt = jnp.sum(lt.astype(jnp.int32), axis=1, keepdims=True)
    need = k - cnt_lt
    eq = (key == t).astype(jnp.int32)
    col = jax.lax.broadcasted_iota(jnp.int32, (rows, n), 1)
    incl = eq
    sh = 1
    while sh < n:
        incl = incl + jnp.where(col >= sh, jnp.roll(incl, sh, axis=1), 0)
        sh *= 2
    prefix_excl = incl - eq
    sel = lt | ((eq > 0) & (prefix_excl < need))
    sel = sel.astype(jnp.int32)
    union = sel[0:b, :]
    for h in range(1, heads):
        union = union | sel[h * b:(h + 1) * b, :]
    out_ref[...] = jnp.where(union > 0, 0, srcmask_ref[...])


def _xcat_block(x_ref, wt_ref, bt_ref, out_ref, *, heads):
    y = jnp.dot(x_ref[0], wt_ref[...],
                preferred_element_type=jnp.float32) + bt_ref[...]
    out_ref[0] = jnp.concatenate([y] * heads, axis=1)


def kernel(adj_list, x, src_mask, W1, b1, W2, b2, Wt, bt):
    heads, b, n, _ = adj_list.shape
    d = x.shape[-1]
    k = int(0.5 * n) + 1
    bm = min(1024, n)
    nb = n // bm

    scores, adj_out = pl.pallas_call(
        functools.partial(_score_block, bm=bm),
        grid=(heads, b, nb),
        in_specs=[
            pl.BlockSpec((1, 1, bm, n), lambda h, bb, i: (h, bb, i, 0)),
            pl.BlockSpec((1, n, d), lambda h, bb, i: (bb, 0, 0)),
            pl.BlockSpec((d, d), lambda h, bb, i: (0, 0)),
            pl.BlockSpec((1, d), lambda h, bb, i: (0, 0)),
            pl.BlockSpec((1, d), lambda h, bb, i: (0, 0)),
            pl.BlockSpec((1, 1), lambda h, bb, i: (0, 0)),
        ],
        out_specs=[
            pl.BlockSpec((1, 1, bm, 1), lambda h, bb, i: (h, bb, i, 0)),
            pl.BlockSpec((1, 1, bm, n), lambda h, bb, i: (h, bb, i, 0)),
        ],
        out_shape=[
            jax.ShapeDtypeStruct((heads, b, nb * bm, 1), jnp.float32),
            jax.ShapeDtypeStruct((heads, b, n, n), jnp.float32),
        ],
        compiler_params=pltpu.CompilerParams(
            dimension_semantics=("parallel", "parallel", "arbitrary")),
    )(adj_list, x, W1, b1.reshape(1, d), W2.reshape(1, d)[:, :],
      b2.reshape(1, 1))

    scores2d = scores.reshape(heads * b, n)
    src2d = src_mask.reshape(b, n).astype(jnp.int32)

    mask2d = pl.pallas_call(
        functools.partial(_select_block, k=k, heads=heads, b=b, n=n),
        out_shape=jax.ShapeDtypeStruct((b, n), jnp.int32),
    )(scores2d, src2d)
    mask_out = mask2d.astype(jnp.bool_).reshape(b, 1, n)

    x_cat = pl.pallas_call(
        functools.partial(_xcat_block, heads=heads),
        grid=(b,),
        in_specs=[
            pl.BlockSpec((1, n, d), lambda bb: (bb, 0, 0)),
            pl.BlockSpec((d, d // heads), lambda bb: (0, 0)),
            pl.BlockSpec((1, d // heads), lambda bb: (0, 0)),
        ],
        out_specs=pl.BlockSpec((1, n, d), lambda bb: (bb, 0, 0)),
        out_shape=jax.ShapeDtypeStruct((b, n, d), jnp.float32),
    )(x, Wt, bt.reshape(1, d // heads))

    return (adj_out, x_cat, mask_out)
